# baseline (speedup 1.0000x reference)
import functools

import jax
import jax.numpy as jnp
from jax import lax
from jax.experimental import pallas as pl
from jax.experimental.pallas import tpu as pltpu
from jax.experimental.pallas import tpu_sc as plsc

VOCAB = 1000000
D = 64
NW = 32
CHUNK = 128
NBUF = 4
LA = 2
PCHUNK = 256
PROWS_LO = 31256
PROWS_HI = VOCAB - 31 * PROWS_LO
PFULL_LO = PROWS_LO // PCHUNK
PFULL_HI = PROWS_HI // PCHUNK
PTAIL_LO = PROWS_LO - PFULL_LO * PCHUNK
PTAIL_HI = PROWS_HI - PFULL_HI * PCHUNK

_mesh = plsc.VectorSubcoreMesh(core_axis_name="c", subcore_axis_name="s")


def _copy_rows(src, dst, n):
    def body(i, _):
        for k in range(4):
            dst[i, pl.ds(16 * k, 16)] = src[i, pl.ds(16 * k, 16)]
        return 0

    lax.fori_loop(0, n, body, 0)


@functools.partial(
    pl.kernel,
    mesh=_mesh,
    out_type=jax.ShapeDtypeStruct((VOCAB, 2 * D), jnp.float32),
    scratch_types=[
        pltpu.VMEM((2, PCHUNK, D), jnp.float32),
        pltpu.VMEM((2, PCHUNK, 2 * D), jnp.float32),
        pltpu.SemaphoreType.DMA((2,)),
        pltpu.SemaphoreType.DMA((2,)),
    ],
)
def _padcopy(tab_hbm, tp_hbm, rbuf, wbuf, rsem, wsem):
    wid = lax.axis_index("s") * 2 + lax.axis_index("c")
    base = jnp.where(wid < 31, wid * PROWS_LO, 31 * PROWS_LO)
    nfull = jnp.where(wid < 31, PFULL_LO, PFULL_HI)

    def rd(j, b):
        pltpu.async_copy(
            tab_hbm.at[pl.ds(base + j * PCHUNK, PCHUNK), :], rbuf.at[b], rsem.at[b]
        )

    rd(0, 0)

    def step(j, _):
        b = lax.rem(j, 2)
        pltpu.make_async_copy(
            tab_hbm.at[pl.ds(base + j * PCHUNK, PCHUNK), :], rbuf.at[b], rsem.at[b]
        ).wait()

        @pl.when(j + 1 < nfull)
        def _():
            rd(j + 1, 1 - b)

        @pl.when(j >= 2)
        def _():
            pltpu.make_async_copy(
                wbuf.at[b], tp_hbm.at[pl.ds(base, PCHUNK), :], wsem.at[b]
            ).wait()

        _copy_rows(rbuf.at[b], wbuf.at[b], PCHUNK)
        pltpu.async_copy(
            wbuf.at[b], tp_hbm.at[pl.ds(base + j * PCHUNK, PCHUNK), :], wsem.at[b]
        )
        return 0

    lax.fori_loop(0, nfull, step, 0)
    for b in range(2):
        @pl.when(nfull >= b + 1)
        def _():
            j = nfull - 1 - b
            pltpu.make_async_copy(
                wbuf.at[1 - b], tp_hbm.at[pl.ds(base, PCHUNK), :], wsem.at[1 - b]
            ).wait()

    t0 = base + nfull * PCHUNK

    @pl.when(wid < 31)
    def _():
        pltpu.sync_copy(
            tab_hbm.at[pl.ds(t0, PTAIL_LO), :], rbuf.at[0, pl.ds(0, PTAIL_LO)]
        )
        _copy_rows(rbuf.at[0], wbuf.at[0], PTAIL_LO)
        pltpu.sync_copy(
            wbuf.at[0, pl.ds(0, PTAIL_LO)], tp_hbm.at[pl.ds(t0, PTAIL_LO), :]
        )

    @pl.when(wid == 31)
    def _():
        pltpu.sync_copy(
            tab_hbm.at[pl.ds(t0, PTAIL_HI), :], rbuf.at[0, pl.ds(0, PTAIL_HI)]
        )
        _copy_rows(rbuf.at[0], wbuf.at[0], PTAIL_HI)
        pltpu.sync_copy(
            wbuf.at[0, pl.ds(0, PTAIL_HI)], tp_hbm.at[pl.ds(t0, PTAIL_HI), :]
        )


def _make_gather(n_chunks):
    @functools.partial(
        pl.kernel,
        mesh=_mesh,
        out_type=jax.ShapeDtypeStruct((NW * n_chunks, CHUNK, D), jnp.float32),
        scratch_types=[
            pltpu.VMEM((n_chunks, CHUNK), jnp.int32),
            pltpu.VMEM((NBUF, CHUNK, 2 * D), jnp.float32),
            pltpu.VMEM((2, CHUNK, D), jnp.float32),
            pltpu.SemaphoreType.DMA((NBUF,)),
            pltpu.SemaphoreType.DMA((2,)),
        ],
    )
    def _gather(x_hbm, tp_hbm, out_hbm, idx_v, rows_v, cbuf, gsem, osem):
        wid = lax.axis_index("s") * 2 + lax.axis_index("c")
        base = wid * n_chunks
        pltpu.sync_copy(x_hbm.at[pl.ds(base, n_chunks), :], idx_v)

        def fire(s, b):
            pltpu.async_copy(tp_hbm.at[idx_v.at[s]], rows_v.at[b], gsem.at[b])

        for b in range(LA):
            fire(b, b)

        def outer(t, _):
            j0 = t * NBUF
            for b in range(NBUF):
                s = j0 + b
                c = b % 2
                pltpu.make_async_copy(
                    tp_hbm.at[idx_v.at[s]], rows_v.at[b], gsem.at[b]
                ).wait()

                @pl.when(s >= 2)
                def _():
                    pltpu.make_async_copy(
                        cbuf.at[c], out_hbm.at[base + s - 2], osem.at[c]
                    ).wait()

                _copy_rows(rows_v.at[b], cbuf.at[c], CHUNK)
                pltpu.async_copy(cbuf.at[c], out_hbm.at[base + s], osem.at[c])

                @pl.when(s + LA < n_chunks)
                def _():
                    fire(s + LA, (b + LA) % NBUF)
            return 0

        lax.fori_loop(0, n_chunks // NBUF, outer, 0)

        for s in (n_chunks - 2, n_chunks - 1):
            c = s % 2
            pltpu.make_async_copy(
                cbuf.at[c], out_hbm.at[base + s], osem.at[c]
            ).wait()

    return _gather


@jax.jit
def _embed(x, table):
    bsz, hist = x.shape
    n_rows = bsz * hist
    n_chunks = n_rows // (NW * CHUNK)
    x2 = x.reshape(n_rows // CHUNK, CHUNK).astype(jnp.int32)
    t_pad = _padcopy(table)
    out = _make_gather(n_chunks)(x2, t_pad)
    return out.reshape(bsz, hist, D)


def kernel(x, table):
    return _embed(x, table)


# speedup vs baseline: 1.0324x; 1.0324x over previous
import functools

import jax
import jax.numpy as jnp
from jax import lax
from jax.experimental import pallas as pl
from jax.experimental.pallas import tpu as pltpu
from jax.experimental.pallas import tpu_sc as plsc

VOCAB = 1000000
D = 64
NW = 32
CHUNK = 128
NBUF = 4
LA = 2
PCHUNK = 256
PROWS_LO = 31256
PROWS_HI = VOCAB - 31 * PROWS_LO
PFULL_LO = PROWS_LO // PCHUNK
PFULL_HI = PROWS_HI // PCHUNK
PTAIL_LO = PROWS_LO - PFULL_LO * PCHUNK
PTAIL_HI = PROWS_HI - PFULL_HI * PCHUNK

_mesh = plsc.VectorSubcoreMesh(core_axis_name="c", subcore_axis_name="s")


def _copy_rows(src, dst, n):
    assert n % 8 == 0

    def body(i, _):
        r0 = i * 8
        for r in range(8):
            for k in range(4):
                dst[r0 + r, pl.ds(16 * k, 16)] = src[r0 + r, pl.ds(16 * k, 16)]
        return 0

    lax.fori_loop(0, n // 8, body, 0)


@functools.partial(
    pl.kernel,
    mesh=_mesh,
    out_type=jax.ShapeDtypeStruct((VOCAB, 2 * D), jnp.float32),
    scratch_types=[
        pltpu.VMEM((2, PCHUNK, D), jnp.float32),
        pltpu.VMEM((2, PCHUNK, 2 * D), jnp.float32),
        pltpu.SemaphoreType.DMA((2,)),
        pltpu.SemaphoreType.DMA((2,)),
    ],
    compiler_params=pltpu.CompilerParams(use_tc_tiling_on_sc=True),
)
def _padcopy(tab_hbm, tp_hbm, rbuf, wbuf, rsem, wsem):
    wid = lax.axis_index("s") * 2 + lax.axis_index("c")
    base = jnp.where(wid < 31, wid * PROWS_LO, 31 * PROWS_LO)
    nfull = jnp.where(wid < 31, PFULL_LO, PFULL_HI)

    def rd(j, b):
        pltpu.async_copy(
            tab_hbm.at[pl.ds(base + j * PCHUNK, PCHUNK), :], rbuf.at[b], rsem.at[b]
        )

    rd(0, 0)

    def step(j, _):
        b = lax.rem(j, 2)
        pltpu.make_async_copy(
            tab_hbm.at[pl.ds(base + j * PCHUNK, PCHUNK), :], rbuf.at[b], rsem.at[b]
        ).wait()

        @pl.when(j + 1 < nfull)
        def _():
            rd(j + 1, 1 - b)

        @pl.when(j >= 2)
        def _():
            pltpu.make_async_copy(
                wbuf.at[b], tp_hbm.at[pl.ds(base, PCHUNK), :], wsem.at[b]
            ).wait()

        _copy_rows(rbuf.at[b], wbuf.at[b], PCHUNK)
        pltpu.async_copy(
            wbuf.at[b], tp_hbm.at[pl.ds(base + j * PCHUNK, PCHUNK), :], wsem.at[b]
        )
        return 0

    lax.fori_loop(0, nfull, step, 0)
    for b in range(2):
        @pl.when(nfull >= b + 1)
        def _():
            j = nfull - 1 - b
            pltpu.make_async_copy(
                wbuf.at[1 - b], tp_hbm.at[pl.ds(base, PCHUNK), :], wsem.at[1 - b]
            ).wait()

    t0 = base + nfull * PCHUNK

    @pl.when(wid < 31)
    def _():
        pltpu.sync_copy(
            tab_hbm.at[pl.ds(t0, PTAIL_LO), :], rbuf.at[0, pl.ds(0, PTAIL_LO)]
        )
        _copy_rows(rbuf.at[0], wbuf.at[0], PTAIL_LO)
        pltpu.sync_copy(
            wbuf.at[0, pl.ds(0, PTAIL_LO)], tp_hbm.at[pl.ds(t0, PTAIL_LO), :]
        )

    @pl.when(wid == 31)
    def _():
        pltpu.sync_copy(
            tab_hbm.at[pl.ds(t0, PTAIL_HI), :], rbuf.at[0, pl.ds(0, PTAIL_HI)]
        )
        _copy_rows(rbuf.at[0], wbuf.at[0], PTAIL_HI)
        pltpu.sync_copy(
            wbuf.at[0, pl.ds(0, PTAIL_HI)], tp_hbm.at[pl.ds(t0, PTAIL_HI), :]
        )


def _make_gather(n_chunks):
    @functools.partial(
        pl.kernel,
        mesh=_mesh,
        out_type=jax.ShapeDtypeStruct((NW * n_chunks * CHUNK, D), jnp.float32),
        scratch_types=[
            pltpu.VMEM((n_chunks, CHUNK), jnp.int32),
            pltpu.VMEM((NBUF, CHUNK, 2 * D), jnp.float32),
            pltpu.VMEM((2, CHUNK, D), jnp.float32),
            pltpu.SemaphoreType.DMA((NBUF,)),
            pltpu.SemaphoreType.DMA((2,)),
        ],
        compiler_params=pltpu.CompilerParams(use_tc_tiling_on_sc=True),
    )
    def _gather(x_hbm, tp_hbm, out_hbm, idx_v, rows_v, cbuf, gsem, osem):
        wid = lax.axis_index("s") * 2 + lax.axis_index("c")
        base = wid * n_chunks
        pltpu.sync_copy(x_hbm.at[pl.ds(base, n_chunks), :], idx_v)

        def fire(s, b):
            pltpu.async_copy(tp_hbm.at[idx_v.at[s]], rows_v.at[b], gsem.at[b])

        for b in range(LA):
            fire(b, b)

        def outer(t, _):
            j0 = t * NBUF
            for b in range(NBUF):
                s = j0 + b
                c = b % 2
                pltpu.make_async_copy(
                    tp_hbm.at[idx_v.at[s]], rows_v.at[b], gsem.at[b]
                ).wait()

                @pl.when(s >= 2)
                def _():
                    pltpu.make_async_copy(
                        cbuf.at[c], out_hbm.at[pl.ds((base + s - 2) * CHUNK, CHUNK), :], osem.at[c]
                    ).wait()

                _copy_rows(rows_v.at[b], cbuf.at[c], CHUNK)
                pltpu.async_copy(cbuf.at[c], out_hbm.at[pl.ds((base + s) * CHUNK, CHUNK), :], osem.at[c])

                @pl.when(s + LA < n_chunks)
                def _():
                    fire(s + LA, (b + LA) % NBUF)
            return 0

        lax.fori_loop(0, n_chunks // NBUF, outer, 0)

        for s in (n_chunks - 2, n_chunks - 1):
            c = s % 2
            pltpu.make_async_copy(
                cbuf.at[c], out_hbm.at[pl.ds((base + s) * CHUNK, CHUNK), :], osem.at[c]
            ).wait()

    return _gather


@jax.jit
def _embed(x, table):
    bsz, hist = x.shape
    n_rows = bsz * hist
    n_chunks = n_rows // (NW * CHUNK)
    x2 = x.reshape(n_rows // CHUNK, CHUNK).astype(jnp.int32)
    t_pad = _padcopy(table)
    out = _make_gather(n_chunks)(x2, t_pad)
    return out.reshape(bsz, hist, D)


def kernel(x, table):
    return _embed(x, table)


# speedup vs baseline: 1.0337x; 1.0013x over previous
import functools

import jax
import jax.numpy as jnp
from jax import lax
from jax.experimental import pallas as pl
from jax.experimental.pallas import tpu as pltpu
from jax.experimental.pallas import tpu_sc as plsc

VOCAB = 1000000
D = 64
NW = 32
CHUNK = 128
NBUF = 4
LA = 2
PCHUNK = 128
PROWS_LO = 31256
PROWS_HI = VOCAB - 31 * PROWS_LO
PFULL_LO = PROWS_LO // PCHUNK
PFULL_HI = PROWS_HI // PCHUNK
PTAIL_LO = PROWS_LO - PFULL_LO * PCHUNK
PTAIL_HI = PROWS_HI - PFULL_HI * PCHUNK

_mesh = plsc.VectorSubcoreMesh(core_axis_name="c", subcore_axis_name="s")


def _copy_rows(src, dst, n):
    assert n % 8 == 0

    def body(i, _):
        r0 = i * 8
        for r in range(8):
            for k in range(4):
                dst[r0 + r, pl.ds(16 * k, 16)] = src[r0 + r, pl.ds(16 * k, 16)]
        return 0

    lax.fori_loop(0, n // 8, body, 0)


@functools.partial(
    pl.kernel,
    mesh=_mesh,
    out_type=jax.ShapeDtypeStruct((VOCAB, 2 * D), jnp.float32),
    scratch_types=[
        pltpu.VMEM((3, PCHUNK, D), jnp.float32),
        pltpu.VMEM((2, PCHUNK, 2 * D), jnp.float32),
        pltpu.SemaphoreType.DMA((3,)),
        pltpu.SemaphoreType.DMA((2,)),
    ],
    compiler_params=pltpu.CompilerParams(use_tc_tiling_on_sc=True),
)
def _padcopy(tab_hbm, tp_hbm, rbuf, wbuf, rsem, wsem):
    wid = lax.axis_index("s") * 2 + lax.axis_index("c")
    base = jnp.where(wid < 31, wid * PROWS_LO, 31 * PROWS_LO)
    nfull = jnp.where(wid < 31, PFULL_LO, PFULL_HI)

    def rd(j, b):
        pltpu.async_copy(
            tab_hbm.at[pl.ds(base + j * PCHUNK, PCHUNK), :], rbuf.at[b], rsem.at[b]
        )

    rd(0, 0)
    rd(1, 1)
    rd(2, 2)

    def step(j, _):
        b = lax.rem(j, 3)
        w = lax.rem(j, 2)
        pltpu.make_async_copy(
            tab_hbm.at[pl.ds(base + j * PCHUNK, PCHUNK), :], rbuf.at[b], rsem.at[b]
        ).wait()

        @pl.when(j >= 2)
        def _():
            pltpu.make_async_copy(
                wbuf.at[w], tp_hbm.at[pl.ds(base, PCHUNK), :], wsem.at[w]
            ).wait()

        _copy_rows(rbuf.at[b], wbuf.at[w], PCHUNK)

        @pl.when(j + 3 < nfull)
        def _():
            rd(j + 3, b)

        pltpu.async_copy(
            wbuf.at[w], tp_hbm.at[pl.ds(base + j * PCHUNK, PCHUNK), :], wsem.at[w]
        )
        return 0

    lax.fori_loop(0, nfull, step, 0)
    for k in range(2):
        w = lax.rem(nfull - 1 - k, 2)
        pltpu.make_async_copy(
            wbuf.at[w], tp_hbm.at[pl.ds(base, PCHUNK), :], wsem.at[w]
        ).wait()

    t0 = base + nfull * PCHUNK

    @pl.when(wid < 31)
    def _():
        pltpu.sync_copy(
            tab_hbm.at[pl.ds(t0, PTAIL_LO), :], rbuf.at[0, pl.ds(0, PTAIL_LO)]
        )
        _copy_rows(rbuf.at[0], wbuf.at[0], PTAIL_LO)
        pltpu.sync_copy(
            wbuf.at[0, pl.ds(0, PTAIL_LO)], tp_hbm.at[pl.ds(t0, PTAIL_LO), :]
        )

    @pl.when(wid == 31)
    def _():
        pltpu.sync_copy(
            tab_hbm.at[pl.ds(t0, PTAIL_HI), :], rbuf.at[0, pl.ds(0, PTAIL_HI)]
        )
        _copy_rows(rbuf.at[0], wbuf.at[0], PTAIL_HI)
        pltpu.sync_copy(
            wbuf.at[0, pl.ds(0, PTAIL_HI)], tp_hbm.at[pl.ds(t0, PTAIL_HI), :]
        )


def _make_gather(n_chunks):
    @functools.partial(
        pl.kernel,
        mesh=_mesh,
        out_type=jax.ShapeDtypeStruct((NW * n_chunks * CHUNK, D), jnp.float32),
        scratch_types=[
            pltpu.VMEM((n_chunks, CHUNK), jnp.int32),
            pltpu.VMEM((NBUF, CHUNK, 2 * D), jnp.float32),
            pltpu.VMEM((2, CHUNK, D), jnp.float32),
            pltpu.SemaphoreType.DMA((NBUF,)),
            pltpu.SemaphoreType.DMA((2,)),
        ],
        compiler_params=pltpu.CompilerParams(use_tc_tiling_on_sc=True),
    )
    def _gather(x_hbm, tp_hbm, out_hbm, idx_v, rows_v, cbuf, gsem, osem):
        wid = lax.axis_index("s") * 2 + lax.axis_index("c")
        base = wid * n_chunks
        pltpu.sync_copy(x_hbm.at[pl.ds(base, n_chunks), :], idx_v)

        def fire(s, b):
            pltpu.async_copy(tp_hbm.at[idx_v.at[s]], rows_v.at[b], gsem.at[b])

        for b in range(LA):
            fire(b, b)

        def outer(t, _):
            j0 = t * NBUF
            for b in range(NBUF):
                s = j0 + b
                c = b % 2
                pltpu.make_async_copy(
                    tp_hbm.at[idx_v.at[s]], rows_v.at[b], gsem.at[b]
                ).wait()

                @pl.when(s >= 2)
                def _():
                    pltpu.make_async_copy(
                        cbuf.at[c], out_hbm.at[pl.ds((base + s - 2) * CHUNK, CHUNK), :], osem.at[c]
                    ).wait()

                _copy_rows(rows_v.at[b], cbuf.at[c], CHUNK)
                pltpu.async_copy(cbuf.at[c], out_hbm.at[pl.ds((base + s) * CHUNK, CHUNK), :], osem.at[c])

                @pl.when(s + LA < n_chunks)
                def _():
                    fire(s + LA, (b + LA) % NBUF)
            return 0

        lax.fori_loop(0, n_chunks // NBUF, outer, 0)

        for s in (n_chunks - 2, n_chunks - 1):
            c = s % 2
            pltpu.make_async_copy(
                cbuf.at[c], out_hbm.at[pl.ds((base + s) * CHUNK, CHUNK), :], osem.at[c]
            ).wait()

    return _gather


@jax.jit
def _embed(x, table):
    bsz, hist = x.shape
    n_rows = bsz * hist
    n_chunks = n_rows // (NW * CHUNK)
    x2 = x.reshape(n_rows // CHUNK, CHUNK).astype(jnp.int32)
    t_pad = _padcopy(table)
    out = _make_gather(n_chunks)(x2, t_pad)
    return out.reshape(bsz, hist, D)


def kernel(x, table):
    return _embed(x, table)


# speedup vs baseline: 1.3371x; 1.2935x over previous
import functools

import jax
import jax.numpy as jnp
from jax import lax
from jax.experimental import pallas as pl
from jax.experimental.pallas import tpu as pltpu
from jax.experimental.pallas import tpu_sc as plsc

VOCAB = 1000000
D = 64
NW = 32
CHUNK = 128
NBUF = 4
LA = 2
PCHUNK = 128
PROWS_LO = 31256
PROWS_HI = VOCAB - 31 * PROWS_LO
PFULL_LO = PROWS_LO // PCHUNK
PFULL_HI = PROWS_HI // PCHUNK
PTAIL_LO = PROWS_LO - PFULL_LO * PCHUNK
PTAIL_HI = PROWS_HI - PFULL_HI * PCHUNK

_mesh = plsc.VectorSubcoreMesh(core_axis_name="c", subcore_axis_name="s")


def _copy_rows(src, dst, n):
    assert n % 8 == 0

    def body(i, _):
        r0 = i * 8
        for r in range(8):
            for k in range(4):
                dst[r0 + r, pl.ds(16 * k, 16)] = src[r0 + r, pl.ds(16 * k, 16)]
        return 0

    lax.fori_loop(0, n // 8, body, 0)


@functools.partial(
    pl.kernel,
    mesh=_mesh,
    out_type=jax.ShapeDtypeStruct((VOCAB, 2 * D), jnp.float32),
    scratch_types=[
        pltpu.VMEM((3, PCHUNK, D), jnp.float32),
        pltpu.VMEM((2, PCHUNK, 2 * D), jnp.float32),
        pltpu.SemaphoreType.DMA((3,)),
        pltpu.SemaphoreType.DMA((2,)),
    ],
    compiler_params=pltpu.CompilerParams(use_tc_tiling_on_sc=True),
)
def _padcopy(tab_hbm, tp_hbm, rbuf, wbuf, rsem, wsem):
    wid = lax.axis_index("s") * 2 + lax.axis_index("c")
    base = jnp.where(wid < 31, wid * PROWS_LO, 31 * PROWS_LO)
    nfull = jnp.where(wid < 31, PFULL_LO, PFULL_HI)

    def rd(j, b):
        pltpu.async_copy(
            tab_hbm.at[pl.ds(base + j * PCHUNK, PCHUNK), :], rbuf.at[b], rsem.at[b]
        )

    rd(0, 0)
    rd(1, 1)
    rd(2, 2)

    def step(j, _):
        b = lax.rem(j, 3)
        w = lax.rem(j, 2)
        pltpu.make_async_copy(
            tab_hbm.at[pl.ds(base + j * PCHUNK, PCHUNK), :], rbuf.at[b], rsem.at[b]
        ).wait()

        @pl.when(j >= 2)
        def _():
            pltpu.make_async_copy(
                wbuf.at[w], tp_hbm.at[pl.ds(base, PCHUNK), :], wsem.at[w]
            ).wait()

        _copy_rows(rbuf.at[b], wbuf.at[w], PCHUNK)

        @pl.when(j + 3 < nfull)
        def _():
            rd(j + 3, b)

        pltpu.async_copy(
            wbuf.at[w], tp_hbm.at[pl.ds(base + j * PCHUNK, PCHUNK), :], wsem.at[w]
        )
        return 0

    lax.fori_loop(0, nfull, step, 0)
    for k in range(2):
        w = lax.rem(nfull - 1 - k, 2)
        pltpu.make_async_copy(
            wbuf.at[w], tp_hbm.at[pl.ds(base, PCHUNK), :], wsem.at[w]
        ).wait()

    t0 = base + nfull * PCHUNK

    @pl.when(wid < 31)
    def _():
        pltpu.sync_copy(
            tab_hbm.at[pl.ds(t0, PTAIL_LO), :], rbuf.at[0, pl.ds(0, PTAIL_LO)]
        )
        _copy_rows(rbuf.at[0], wbuf.at[0], PTAIL_LO)
        pltpu.sync_copy(
            wbuf.at[0, pl.ds(0, PTAIL_LO)], tp_hbm.at[pl.ds(t0, PTAIL_LO), :]
        )

    @pl.when(wid == 31)
    def _():
        pltpu.sync_copy(
            tab_hbm.at[pl.ds(t0, PTAIL_HI), :], rbuf.at[0, pl.ds(0, PTAIL_HI)]
        )
        _copy_rows(rbuf.at[0], wbuf.at[0], PTAIL_HI)
        pltpu.sync_copy(
            wbuf.at[0, pl.ds(0, PTAIL_HI)], tp_hbm.at[pl.ds(t0, PTAIL_HI), :]
        )


def _make_gather(n_chunks):
    @functools.partial(
        pl.kernel,
        mesh=_mesh,
        out_type=jax.ShapeDtypeStruct((NW * n_chunks * CHUNK, D), jnp.float32),
        scratch_types=[
            pltpu.VMEM((n_chunks, CHUNK), jnp.int32),
            pltpu.VMEM((NBUF, CHUNK, 2 * D), jnp.float32),
            pltpu.VMEM((2, CHUNK, D), jnp.float32),
            pltpu.SemaphoreType.DMA((NBUF,)),
            pltpu.SemaphoreType.DMA((2,)),
        ],
        compiler_params=pltpu.CompilerParams(use_tc_tiling_on_sc=True),
    )
    def _gather(x_hbm, tp_hbm, out_hbm, idx_v, rows_v, cbuf, gsem, osem):
        wid = lax.axis_index("s") * 2 + lax.axis_index("c")
        base = wid * n_chunks
        pltpu.sync_copy(x_hbm.at[pl.ds(base, n_chunks), :], idx_v)

        def fire(s, b):
            pltpu.async_copy(tp_hbm.at[idx_v.at[s]], rows_v.at[b], gsem.at[b])

        for b in range(LA):
            fire(b, b)

        def outer(t, _):
            j0 = t * NBUF
            for b in range(NBUF):
                s = j0 + b
                c = b % 2
                pltpu.make_async_copy(
                    tp_hbm.at[idx_v.at[s]], rows_v.at[b], gsem.at[b]
                ).wait()

                @pl.when(s >= 2)
                def _():
                    pltpu.make_async_copy(
                        cbuf.at[c], out_hbm.at[pl.ds((base + s - 2) * CHUNK, CHUNK), :], osem.at[c]
                    ).wait()

                _copy_rows(rows_v.at[b], cbuf.at[c], CHUNK)
                pltpu.async_copy(cbuf.at[c], out_hbm.at[pl.ds((base + s) * CHUNK, CHUNK), :], osem.at[c])

                @pl.when(s + LA < n_chunks)
                def _():
                    fire(s + LA, (b + LA) % NBUF)
            return 0

        lax.fori_loop(0, n_chunks // NBUF, outer, 0)

        for s in (n_chunks - 2, n_chunks - 1):
            c = s % 2
            pltpu.make_async_copy(
                cbuf.at[c], out_hbm.at[pl.ds((base + s) * CHUNK, CHUNK), :], osem.at[c]
            ).wait()

    return _gather


@jax.jit
def _embed(x, table):
    bsz, hist = x.shape
    n_rows = bsz * hist
    n_chunks = n_rows // (NW * CHUNK)
    x2 = x.reshape(n_rows // CHUNK, CHUNK).astype(jnp.int32)
    t_pad = jnp.pad(table, ((0, 0), (0, D)))
    out = _make_gather(n_chunks)(x2, t_pad)
    return out.reshape(bsz, hist, D)


def kernel(x, table):
    return _embed(x, table)


# speedup vs baseline: 1.3389x; 1.0013x over previous
import functools

import jax
import jax.numpy as jnp
from jax import lax
from jax.experimental import pallas as pl
from jax.experimental.pallas import tpu as pltpu
from jax.experimental.pallas import tpu_sc as plsc

VOCAB = 1000000
D = 64
NW = 32
CHUNK = 128
NBUF = 4
LA = 3
PCHUNK = 128
PROWS_LO = 31256
PROWS_HI = VOCAB - 31 * PROWS_LO
PFULL_LO = PROWS_LO // PCHUNK
PFULL_HI = PROWS_HI // PCHUNK
PTAIL_LO = PROWS_LO - PFULL_LO * PCHUNK
PTAIL_HI = PROWS_HI - PFULL_HI * PCHUNK

_mesh = plsc.VectorSubcoreMesh(core_axis_name="c", subcore_axis_name="s")


def _copy_rows(src, dst, n):
    assert n % 8 == 0

    def body(i, _):
        r0 = i * 8
        for r in range(8):
            for k in range(4):
                dst[r0 + r, pl.ds(16 * k, 16)] = src[r0 + r, pl.ds(16 * k, 16)]
        return 0

    lax.fori_loop(0, n // 8, body, 0)


@functools.partial(
    pl.kernel,
    mesh=_mesh,
    out_type=jax.ShapeDtypeStruct((VOCAB, 2 * D), jnp.float32),
    scratch_types=[
        pltpu.VMEM((3, PCHUNK, D), jnp.float32),
        pltpu.VMEM((2, PCHUNK, 2 * D), jnp.float32),
        pltpu.SemaphoreType.DMA((3,)),
        pltpu.SemaphoreType.DMA((2,)),
    ],
    compiler_params=pltpu.CompilerParams(use_tc_tiling_on_sc=True),
)
def _padcopy(tab_hbm, tp_hbm, rbuf, wbuf, rsem, wsem):
    wid = lax.axis_index("s") * 2 + lax.axis_index("c")
    base = jnp.where(wid < 31, wid * PROWS_LO, 31 * PROWS_LO)
    nfull = jnp.where(wid < 31, PFULL_LO, PFULL_HI)

    def rd(j, b):
        pltpu.async_copy(
            tab_hbm.at[pl.ds(base + j * PCHUNK, PCHUNK), :], rbuf.at[b], rsem.at[b]
        )

    rd(0, 0)
    rd(1, 1)
    rd(2, 2)

    def step(j, _):
        b = lax.rem(j, 3)
        w = lax.rem(j, 2)
        pltpu.make_async_copy(
            tab_hbm.at[pl.ds(base + j * PCHUNK, PCHUNK), :], rbuf.at[b], rsem.at[b]
        ).wait()

        @pl.when(j >= 2)
        def _():
            pltpu.make_async_copy(
                wbuf.at[w], tp_hbm.at[pl.ds(base, PCHUNK), :], wsem.at[w]
            ).wait()

        _copy_rows(rbuf.at[b], wbuf.at[w], PCHUNK)

        @pl.when(j + 3 < nfull)
        def _():
            rd(j + 3, b)

        pltpu.async_copy(
            wbuf.at[w], tp_hbm.at[pl.ds(base + j * PCHUNK, PCHUNK), :], wsem.at[w]
        )
        return 0

    lax.fori_loop(0, nfull, step, 0)
    for k in range(2):
        w = lax.rem(nfull - 1 - k, 2)
        pltpu.make_async_copy(
            wbuf.at[w], tp_hbm.at[pl.ds(base, PCHUNK), :], wsem.at[w]
        ).wait()

    t0 = base + nfull * PCHUNK

    @pl.when(wid < 31)
    def _():
        pltpu.sync_copy(
            tab_hbm.at[pl.ds(t0, PTAIL_LO), :], rbuf.at[0, pl.ds(0, PTAIL_LO)]
        )
        _copy_rows(rbuf.at[0], wbuf.at[0], PTAIL_LO)
        pltpu.sync_copy(
            wbuf.at[0, pl.ds(0, PTAIL_LO)], tp_hbm.at[pl.ds(t0, PTAIL_LO), :]
        )

    @pl.when(wid == 31)
    def _():
        pltpu.sync_copy(
            tab_hbm.at[pl.ds(t0, PTAIL_HI), :], rbuf.at[0, pl.ds(0, PTAIL_HI)]
        )
        _copy_rows(rbuf.at[0], wbuf.at[0], PTAIL_HI)
        pltpu.sync_copy(
            wbuf.at[0, pl.ds(0, PTAIL_HI)], tp_hbm.at[pl.ds(t0, PTAIL_HI), :]
        )


def _make_gather(n_chunks):
    @functools.partial(
        pl.kernel,
        mesh=_mesh,
        out_type=jax.ShapeDtypeStruct((NW * n_chunks * CHUNK, D), jnp.float32),
        scratch_types=[
            pltpu.VMEM((n_chunks, CHUNK), jnp.int32),
            pltpu.VMEM((NBUF, CHUNK, 2 * D), jnp.float32),
            pltpu.VMEM((2, CHUNK, D), jnp.float32),
            pltpu.SemaphoreType.DMA((NBUF,)),
            pltpu.SemaphoreType.DMA((2,)),
        ],
        compiler_params=pltpu.CompilerParams(use_tc_tiling_on_sc=True),
    )
    def _gather(x_hbm, tp_hbm, out_hbm, idx_v, rows_v, cbuf, gsem, osem):
        wid = lax.axis_index("s") * 2 + lax.axis_index("c")
        base = wid * n_chunks
        pltpu.sync_copy(x_hbm.at[pl.ds(base, n_chunks), :], idx_v)

        def fire(s, b):
            pltpu.async_copy(tp_hbm.at[idx_v.at[s]], rows_v.at[b], gsem.at[b])

        for b in range(LA):
            fire(b, b)

        def outer(t, _):
            j0 = t * NBUF
            for b in range(NBUF):
                s = j0 + b
                c = b % 2
                pltpu.make_async_copy(
                    tp_hbm.at[idx_v.at[s]], rows_v.at[b], gsem.at[b]
                ).wait()

                @pl.when(s >= 2)
                def _():
                    pltpu.make_async_copy(
                        cbuf.at[c], out_hbm.at[pl.ds((base + s - 2) * CHUNK, CHUNK), :], osem.at[c]
                    ).wait()

                _copy_rows(rows_v.at[b], cbuf.at[c], CHUNK)
                pltpu.async_copy(cbuf.at[c], out_hbm.at[pl.ds((base + s) * CHUNK, CHUNK), :], osem.at[c])

                @pl.when(s + LA < n_chunks)
                def _():
                    fire(s + LA, (b + LA) % NBUF)
            return 0

        lax.fori_loop(0, n_chunks // NBUF, outer, 0)

        for s in (n_chunks - 2, n_chunks - 1):
            c = s % 2
            pltpu.make_async_copy(
                cbuf.at[c], out_hbm.at[pl.ds((base + s) * CHUNK, CHUNK), :], osem.at[c]
            ).wait()

    return _gather


@jax.jit
def _embed(x, table):
    bsz, hist = x.shape
    n_rows = bsz * hist
    n_chunks = n_rows // (NW * CHUNK)
    x2 = x.reshape(n_rows // CHUNK, CHUNK).astype(jnp.int32)
    t_pad = jnp.pad(table, ((0, 0), (0, D)))
    out = _make_gather(n_chunks)(x2, t_pad)
    return out.reshape(bsz, hist, D)


def kernel(x, table):
    return _embed(x, table)
